# baseline (speedup 1.0000x reference)
import functools

import jax
import jax.numpy as jnp
from jax import lax
from jax.experimental import pallas as pl
from jax.experimental.pallas import tpu as pltpu
from jax.experimental.pallas import tpu_sc as plsc

VOCAB = 1000000
EMBED_DIM = 64
BATCH = 4096
HIST = 200

NC = 2
NS = 16
NW = NC * NS
B_PER_W = BATCH // NW


def _sc_pool_body(x_hbm, table_hbm, out_hbm, idx0, idx1, rows_v, pooled_v,
                  isem0, isem1, gsem0, gsem1):
    idxs = (idx0, idx1)
    isems = (isem0, isem1)
    gsems = (gsem0, gsem1)
    wid = lax.axis_index("s") * NC + lax.axis_index("c")
    base = wid * B_PER_W

    def issue_idx(row, b):
        pltpu.async_copy(x_hbm.at[pl.ds((base + row) * HIST, HIST)],
                         idxs[b], isems[b])

    def wait_idx(b):
        pltpu.make_async_copy(x_hbm.at[pl.ds(0, HIST)], idxs[b],
                              isems[b]).wait()

    def issue_gather(row_unused, b):
        pltpu.async_copy(table_hbm.at[idxs[b]], rows_v.at[b], gsems[b])

    def wait_gather(b):
        pltpu.make_async_copy(table_hbm.at[pl.ds(0, HIST)], rows_v.at[b],
                              gsems[b]).wait()

    issue_idx(0, 0)
    issue_idx(1, 1)
    wait_idx(0)
    issue_gather(0, 0)

    def outer(i, carry):
        for b in range(2):
            row = i * 2 + b
            wait_gather(b)

            @pl.when(row + 2 < B_PER_W)
            def _():
                issue_idx(row + 2, b)

            def red(l, accs):
                return tuple(
                    accs[j] + rows_v[b, l, pl.ds(j * 16, 16)]
                    for j in range(EMBED_DIM // 16))

            accs = lax.fori_loop(
                0, HIST, red,
                tuple(jnp.zeros((16,), jnp.float32)
                      for _ in range(EMBED_DIM // 16)))
            for j in range(EMBED_DIM // 16):
                pooled_v[row, pl.ds(j * 16, 16)] = accs[j] * (1.0 / HIST)

            @pl.when(row + 1 < B_PER_W)
            def _():
                wait_idx(1 - b)
                issue_gather(row + 1, 1 - b)
        return carry

    lax.fori_loop(0, B_PER_W // 2, outer, 0)
    pltpu.sync_copy(pooled_v, out_hbm.at[pl.ds(base, B_PER_W)])


@jax.jit
def _sc_pool(x_flat, table):
    mesh = plsc.VectorSubcoreMesh(core_axis_name="c", subcore_axis_name="s")
    f = functools.partial(
        pl.kernel,
        out_type=jax.ShapeDtypeStruct((BATCH, EMBED_DIM), jnp.float32),
        mesh=mesh,
        compiler_params=pltpu.CompilerParams(use_tc_tiling_on_sc=False),
        scratch_types=[
            pltpu.VMEM((HIST,), jnp.int32),
            pltpu.VMEM((HIST,), jnp.int32),
            pltpu.VMEM((2, HIST, EMBED_DIM), jnp.float32),
            pltpu.VMEM((B_PER_W, EMBED_DIM), jnp.float32),
            pltpu.SemaphoreType.DMA,
            pltpu.SemaphoreType.DMA,
            pltpu.SemaphoreType.DMA,
            pltpu.SemaphoreType.DMA,
        ],
    )(_sc_pool_body)
    return f(x_flat, table)


def _mlp_body(p_ref, w1_ref, b1_ref, w2_ref, b2_ref, o_ref):
    p = p_ref[...]
    h = lax.dot_general(p, w1_ref[...], (((1,), (1,)), ((), ())),
                        precision=lax.Precision.HIGHEST,
                        preferred_element_type=jnp.float32)
    h = jnp.maximum(h + b1_ref[...], 0.0)
    o_ref[...] = jnp.sum(h * w2_ref[...], axis=1, keepdims=True) + b2_ref[...]


@jax.jit
def _mlp(pooled, W1, b1, W2, b2):
    return pl.pallas_call(
        _mlp_body,
        out_shape=jax.ShapeDtypeStruct((BATCH, 1), jnp.float32),
    )(pooled, W1, b1.reshape(1, 256), W2, b2.reshape(1, 1))


def kernel(x, table, W1, b1, W2, b2):
    x_flat = x.astype(jnp.int32).reshape(BATCH * HIST)
    pooled = _sc_pool(x_flat, table)
    return _mlp(pooled, W1, b1, W2, b2)


# speedup vs baseline: 1.1149x; 1.1149x over previous
import functools

import jax
import jax.numpy as jnp
from jax import lax
from jax.experimental import pallas as pl
from jax.experimental.pallas import tpu as pltpu
from jax.experimental.pallas import tpu_sc as plsc

VOCAB = 1000000
EMBED_DIM = 64
BATCH = 4096
HIST = 200

NC = 2
NS = 16
NW = NC * NS
B_PER_W = BATCH // NW


ROWS_PER_CHUNK = 4
CHUNK_IDX = ROWS_PER_CHUNK * HIST
N_CHUNKS = B_PER_W // ROWS_PER_CHUNK


def _sc_pool_body(x_hbm, table_hbm, out_hbm, idx0, idx1, rows_v, pooled_v,
                  isem0, isem1, gsem0, gsem1):
    idxs = (idx0, idx1)
    isems = (isem0, isem1)
    gsems = (gsem0, gsem1)
    wid = lax.axis_index("s") * NC + lax.axis_index("c")
    base = wid * B_PER_W

    def issue_idx(c, b):
        pltpu.async_copy(
            x_hbm.at[pl.ds((base + c * ROWS_PER_CHUNK) * HIST, CHUNK_IDX)],
            idxs[b], isems[b])

    def wait_idx(b):
        pltpu.make_async_copy(x_hbm.at[pl.ds(0, CHUNK_IDX)], idxs[b],
                              isems[b]).wait()

    def issue_gather(b):
        pltpu.async_copy(table_hbm.at[idxs[b]], rows_v.at[b], gsems[b])

    def wait_gather(b):
        pltpu.make_async_copy(table_hbm.at[pl.ds(0, CHUNK_IDX)],
                              rows_v.at[b], gsems[b]).wait()

    issue_idx(0, 0)
    issue_idx(1, 1)
    wait_idx(0)
    issue_gather(0)

    def outer(i, carry):
        for b in range(2):
            c = i * 2 + b
            wait_gather(b)

            @pl.when(c + 2 < N_CHUNKS)
            def _():
                issue_idx(c + 2, b)

            for r in range(ROWS_PER_CHUNK):
                def red(l, accs, _r=r):
                    return tuple(
                        accs[j] + rows_v[b, _r * HIST + l, pl.ds(j * 16, 16)]
                        for j in range(EMBED_DIM // 16))

                accs = lax.fori_loop(
                    0, HIST, red,
                    tuple(jnp.zeros((16,), jnp.float32)
                          for _ in range(EMBED_DIM // 16)),
                    unroll=8)
                row = c * ROWS_PER_CHUNK + r
                for j in range(EMBED_DIM // 16):
                    pooled_v[row, pl.ds(j * 16, 16)] = accs[j] * (1.0 / HIST)

            @pl.when(c + 1 < N_CHUNKS)
            def _():
                wait_idx(1 - b)
                issue_gather(1 - b)
        return carry

    lax.fori_loop(0, N_CHUNKS // 2, outer, 0)
    pltpu.sync_copy(pooled_v, out_hbm.at[pl.ds(base, B_PER_W)])


@jax.jit
def _sc_pool(x_flat, table):
    mesh = plsc.VectorSubcoreMesh(core_axis_name="c", subcore_axis_name="s")
    f = functools.partial(
        pl.kernel,
        out_type=jax.ShapeDtypeStruct((BATCH, EMBED_DIM), jnp.float32),
        mesh=mesh,
        compiler_params=pltpu.CompilerParams(use_tc_tiling_on_sc=False),
        scratch_types=[
            pltpu.VMEM((CHUNK_IDX,), jnp.int32),
            pltpu.VMEM((CHUNK_IDX,), jnp.int32),
            pltpu.VMEM((2, CHUNK_IDX, EMBED_DIM), jnp.float32),
            pltpu.VMEM((B_PER_W, EMBED_DIM), jnp.float32),
            pltpu.SemaphoreType.DMA,
            pltpu.SemaphoreType.DMA,
            pltpu.SemaphoreType.DMA,
            pltpu.SemaphoreType.DMA,
        ],
    )(_sc_pool_body)
    return f(x_flat, table)


def _mlp_body(p_ref, w1_ref, b1_ref, w2_ref, b2_ref, o_ref):
    p = p_ref[...]
    h = lax.dot_general(p, w1_ref[...], (((1,), (1,)), ((), ())),
                        precision=lax.Precision.HIGHEST,
                        preferred_element_type=jnp.float32)
    h = jnp.maximum(h + b1_ref[...], 0.0)
    o_ref[...] = jnp.sum(h * w2_ref[...], axis=1, keepdims=True) + b2_ref[...]


@jax.jit
def _mlp(pooled, W1, b1, W2, b2):
    return pl.pallas_call(
        _mlp_body,
        out_shape=jax.ShapeDtypeStruct((BATCH, 1), jnp.float32),
    )(pooled, W1, b1.reshape(1, 256), W2, b2.reshape(1, 1))


def kernel(x, table, W1, b1, W2, b2):
    x_flat = x.astype(jnp.int32).reshape(BATCH * HIST)
    pooled = _sc_pool(x_flat, table)
    return _mlp(pooled, W1, b1, W2, b2)
